# baseline (speedup 1.0000x reference)
import functools

import jax
import jax.numpy as jnp
from jax import lax
from jax.experimental import pallas as pl
from jax.experimental.pallas import tpu as pltpu
from jax.experimental.pallas import tpu_sc as plsc

N = 10000
E = 320000
H = 64
NSTEPS = 3

NC, NS = 2, 16
NW = NC * NS
CH = 80
EW = E // NW
NCH = EW // CH
EH = E // 2
EWH = EH // NW
CH2 = 40
NCH2 = EWH // CH2
NPAD = 10240
NPT = NPAD // NS

BE = 4000
BT = 6400
BN = 2000

_f32 = jnp.float32


def _ln(h, g, b):
    mu = jnp.mean(h, axis=-1, keepdims=True)
    d = h - mu
    var = jnp.mean(d * d, axis=-1, keepdims=True)
    return d / jnp.sqrt(var + 1e-5) * g + b


def _dot(a, b):
    return jnp.dot(a, b, preferred_element_type=_f32)


def _full(shape):
    return pl.BlockSpec(shape, lambda i: tuple(0 for _ in shape))


def _rows(bshape):
    return pl.BlockSpec(bshape, lambda i: (i,) + tuple(0 for _ in bshape[1:]))


def _enc_node_body(x, W1, b1, W2, b2, g, b, Wse, Wsh, Wde, Wdh, Wxe,
                   o_enc, o_T0, o_Ax, o_Bx, o_Nx):
    h = jnp.maximum(_dot(x[...], W1[...]) + b1[...], 0.0)
    e = _ln(_dot(h, W2[...]) + b2[...], g[...], b[...])
    o_enc[...] = e
    ax = _dot(e, Wse[...])
    bx = _dot(e, Wde[...])
    o_Ax[...] = ax
    o_Bx[...] = bx
    o_T0[...] = jnp.concatenate(
        [ax + _dot(e, Wsh[...]), bx + _dot(e, Wdh[...])], axis=-1)
    o_Nx[...] = _dot(e, Wxe[...])


def _enc_node_call(x, args):
    outs = [jax.ShapeDtypeStruct((N, H), _f32),
            jax.ShapeDtypeStruct((N, 2 * H), _f32)] \
        + [jax.ShapeDtypeStruct((N, H), _f32)] * 3
    return pl.pallas_call(
        _enc_node_body,
        grid=(N // BN,),
        in_specs=[_rows((BN, 128))] + [_full(a.shape) for a in args],
        out_specs=[_rows((BN, H)), _rows((BN, 2 * H))] + [_rows((BN, H))] * 3,
        out_shape=outs,
    )(x, *args)


def _enc_edge_body(ea_t, W1, b1, W2, b2, g, b, Wee, Weh, o_heP, o_C):
    h = jnp.maximum(
        lax.dot_general(ea_t[...], W1[...], (((0,), (0,)), ((), ())),
                        preferred_element_type=_f32) + b1[...], 0.0)
    e = _ln(_dot(h, W2[...]) + b2[...], g[...], b[...])
    o_heP[...] = jnp.concatenate([e, _dot(e, Weh[...])], axis=-1)
    o_C[...] = _dot(e, Wee[...])


def _enc_edge_call(ea_t, args, half):
    off = half * (EH // BT)
    return pl.pallas_call(
        _enc_edge_body,
        grid=(EH // BT,),
        in_specs=[pl.BlockSpec((16, BT), lambda i, o=off: (0, i + o))]
        + [_full(a.shape) for a in args],
        out_specs=[_rows((BT, 2 * H)), _rows((BT, H))],
        out_shape=[jax.ShapeDtypeStruct((EH, 2 * H), _f32),
                   jax.ShapeDtypeStruct((EH, H), _f32)],
    )(ea_t, *args)


def _u_enc_body(u, W1, b1, W2, b2, g, b, Weu_e, Weu_h, b1e, Wxu_e, Wxu_h, b1x,
                o_encu, o_ue, o_ux):
    h = jnp.maximum(_dot(u[...], W1[...]) + b1[...], 0.0)
    eu = _ln(_dot(h, W2[...]) + b2[...], g[...], b[...])
    o_encu[...] = eu
    o_ue[...] = _dot(eu, Weu_e[...]) + _dot(eu, Weu_h[...]) + b1e[...]
    o_ux[...] = _dot(eu, Wxu_e[...]) + _dot(eu, Wxu_h[...]) + b1x[...]


def _u_enc_call(u, args):
    return pl.pallas_call(
        _u_enc_body,
        out_shape=[jax.ShapeDtypeStruct((1, H), _f32)] * 3,
    )(u, *args)


def _edge_core_body(S, C, P, ue, Weh, W2, b2, g, b, o_heP, o_esum):
    h1 = S[...].astype(_f32) + C[...] + P[...][:, H:] + ue[...]
    out = _ln(_dot(jnp.maximum(h1, 0.0), W2[...]) + b2[...], g[...], b[...])
    o_heP[...] = jnp.concatenate([out, _dot(out, Weh[...])], axis=-1)

    @pl.when(pl.program_id(0) == 0)
    def _():
        o_esum[...] = jnp.zeros_like(o_esum)

    o_esum[...] += jnp.sum(out, axis=0, keepdims=True)


def _edge_core_call(S, C, heP, ue, args):
    return pl.pallas_call(
        _edge_core_body,
        grid=(EH // BE,),
        in_specs=[_rows((BE, H))] * 2
        + [_rows((BE, 2 * H)), _full((1, H))]
        + [_full(a.shape) for a in args],
        out_specs=[_rows((BE, 2 * H)), _full((1, H))],
        out_shape=[jax.ShapeDtypeStruct((EH, 2 * H), _f32),
                   jax.ShapeDtypeStruct((1, H), _f32)],
    )(S, C, heP, ue, *args)


def _node_core_body(ns, ns1, cnt, hx, Nx, Ax, Bx, ux, Wxh, Wxa, W2, b2, g, b,
                    Wsh, Wdh, o_hx, o_T, o_xsum):
    s = ns[0, :, :H] + ns[1, :, :H] + ns1[0, :, :H] + ns1[1, :, :H]
    c = cnt[0, :, 0:1] + cnt[1, :, 0:1]
    eagg = s / jnp.maximum(c, 1.0)
    h1 = Nx[...] + _dot(hx[...], Wxh[...]) + _dot(eagg, Wxa[...]) + ux[...]
    hxn = _ln(_dot(jnp.maximum(h1, 0.0), W2[...]) + b2[...], g[...], b[...])
    o_hx[...] = hxn
    o_T[...] = jnp.concatenate(
        [Ax[...] + _dot(hxn, Wsh[...]), Bx[...] + _dot(hxn, Wdh[...])],
        axis=-1)

    @pl.when(pl.program_id(0) == 0)
    def _():
        o_xsum[...] = jnp.zeros_like(o_xsum)

    o_xsum[...] += jnp.sum(hxn, axis=0, keepdims=True)


def _node_core_call(ns, ns1, cnt, hx, Nx, Ax, Bx, ux, args):
    return pl.pallas_call(
        _node_core_body,
        grid=(N // BN,),
        in_specs=[pl.BlockSpec((2, BN, 2 * H), lambda i: (0, i, 0)),
                  pl.BlockSpec((2, BN, 2 * H), lambda i: (0, i, 0)),
                  pl.BlockSpec((2, BN, 128), lambda i: (0, i, 0))]
        + [_rows((BN, H))] * 4 + [_full((1, H))]
        + [_full(a.shape) for a in args],
        out_specs=[_rows((BN, H)), _rows((BN, 2 * H)), _full((1, H))],
        out_shape=[jax.ShapeDtypeStruct((N, H), _f32),
                   jax.ShapeDtypeStruct((N, 2 * H), _f32),
                   jax.ShapeDtypeStruct((1, H), _f32)],
    )(ns, ns1, cnt, hx, Nx, Ax, Bx, ux, *args)


def _global_body(xsum, esum, esum1, encu, hu, Wu_x, Wu_e, Wu_ue, Wu_uh, b1u,
                 W2u, b2u, gu, bu, Weu_e, Weu_h, b1e, Wxu_e, Wxu_h, b1x,
                 Wd1, bd1, Wd2, bd2, gd, bd,
                 o_hu, o_ue, o_ux, o_uout):
    xa = xsum[...] * (1.0 / N)
    ea = (esum[...] + esum1[...]) * (1.0 / E)
    h1 = (_dot(xa, Wu_x[...]) + _dot(ea, Wu_e[...]) + _dot(encu[...], Wu_ue[...])
          + _dot(hu[...], Wu_uh[...]) + b1u[...])
    hun = _ln(_dot(jnp.maximum(h1, 0.0), W2u[...]) + b2u[...], gu[...], bu[...])
    o_hu[...] = hun
    o_ue[...] = _dot(encu[...], Weu_e[...]) + _dot(hun, Weu_h[...]) + b1e[...]
    o_ux[...] = _dot(encu[...], Wxu_e[...]) + _dot(hun, Wxu_h[...]) + b1x[...]
    hd = jnp.maximum(_dot(hun, Wd1[...]) + bd1[...], 0.0)
    o_uout[...] = _ln(_dot(hd, Wd2[...]) + bd2[...], gd[...], bd[...])


def _global_call(xsum, esum, esum1, encu, hu, args):
    return pl.pallas_call(
        _global_body,
        out_shape=[jax.ShapeDtypeStruct((1, H), _f32)] * 3
        + [jax.ShapeDtypeStruct((1, 16), _f32)],
    )(xsum, esum, esum1, encu, hu, *args)


def _dec_body(h, W1, b1, W2, b2, g, b, o):
    a = jnp.maximum(_dot(h[...][:, :H], W1[...]) + b1[...], 0.0)
    o[...] = _ln(_dot(a, W2[...]) + b2[...], g[...], b[...])


def _dec_e_body(h, W1, b1, W2, b2, g, b, o):
    a = jnp.maximum(_dot(h[...][:, :H], W1[...]) + b1[...], 0.0)
    ot = lax.dot_general(W2[...], a, (((0,), (1,)), ((), ())),
                         preferred_element_type=_f32) + b2[...]
    mu = jnp.mean(ot, axis=0, keepdims=True)
    d = ot - mu
    var = jnp.mean(d * d, axis=0, keepdims=True)
    o[...] = d / jnp.sqrt(var + 1e-5) * g[...] + b[...]


def _dec_e_call(h, args):
    return pl.pallas_call(
        _dec_e_body,
        grid=(EH // BT,),
        in_specs=[_rows((BT, 2 * H))] + [_full(a.shape) for a in args],
        out_specs=pl.BlockSpec((16, BT), lambda i: (0, i)),
        out_shape=jax.ShapeDtypeStruct((16, EH), _f32),
    )(h, *args)


def _dec_call(h, args, rows, brows, fout):
    return pl.pallas_call(
        _dec_body,
        grid=(rows // brows,),
        in_specs=[_rows((brows, h.shape[1]))] + [_full(a.shape) for a in args],
        out_specs=_rows((brows, fout)),
        out_shape=jax.ShapeDtypeStruct((rows, fout), _f32),
    )(h, *args)


def _mesh():
    return plsc.VectorSubcoreMesh(core_axis_name="c", subcore_axis_name="s",
                                  num_cores=NC, num_subcores=NS)


@functools.lru_cache(maxsize=None)
def _sc_gather_kernel(etot, ew, nch, ch):
    def body(t_h, row_h, col_h, s_h, idxa, idxb, ba0, bb0, ba1, bb1,
             sb0, sb1, ga0, gb0, ga1, gb1, w0, w1):
        c = lax.axis_index("c")
        s = lax.axis_index("s")
        w = c * NS + s
        pltpu.sync_copy(row_h.at[w], idxa)
        pltpu.sync_copy(col_h.at[w], idxb)

        def start(j, bufa, bufb, sa, sb):
            pltpu.async_copy(t_h.at[idxa.at[j]], bufa, sa)
            pltpu.async_copy(t_h.at[idxb.at[j]], bufb, sb)

        def finish(j, bufa, bufb, sa, sb, sbuf, ws):
            pltpu.make_async_copy(t_h.at[pl.ds(0, ch)], bufa, sa).wait()
            pltpu.make_async_copy(t_h.at[pl.ds(0, ch)], bufb, sb).wait()

            @pl.when(j >= 2)
            def _():
                pltpu.make_async_copy(sbuf, s_h.at[pl.ds(0, ch), :], ws).wait()

            def add(r, carry):
                for l in range(H // 16):
                    va = bufa[r, pl.ds(l * 16, 16)]
                    vb = bufb[r, pl.ds(H + l * 16, 16)]
                    sbuf[r, pl.ds(l * 16, 16)] = va + vb
                return carry

            lax.fori_loop(0, ch, add, 0)
            pltpu.async_copy(sbuf, s_h.at[pl.ds(w * ew + j * ch, ch), :], ws)

        start(0, ba0, bb0, ga0, gb0)

        def loop(k, carry):
            j0 = 2 * k
            start(j0 + 1, ba1, bb1, ga1, gb1)
            finish(j0, ba0, bb0, ga0, gb0, sb0, w0)

            @pl.when(j0 + 2 < nch)
            def _():
                start(j0 + 2, ba0, bb0, ga0, gb0)

            finish(j0 + 1, ba1, bb1, ga1, gb1, sb1, w1)
            return carry

        lax.fori_loop(0, nch // 2, loop, 0)
        finish(nch - 1, ba0, bb0, ga0, gb0, sb0, w0)
        pltpu.make_async_copy(sb0, s_h.at[pl.ds(0, ch), :], w0).wait()
        pltpu.make_async_copy(sb1, s_h.at[pl.ds(0, ch), :], w1).wait()

    return functools.partial(
        pl.kernel,
        out_type=jax.ShapeDtypeStruct((etot, H), _f32),
        mesh=_mesh(),
        scratch_types=[
            pltpu.VMEM((nch, ch), jnp.int32),
            pltpu.VMEM((nch, ch), jnp.int32),
            pltpu.VMEM((ch, 2 * H), _f32),
            pltpu.VMEM((ch, 2 * H), _f32),
            pltpu.VMEM((ch, 2 * H), _f32),
            pltpu.VMEM((ch, 2 * H), _f32),
            pltpu.VMEM((ch, H), _f32),
            pltpu.VMEM((ch, H), _f32),
        ] + [pltpu.SemaphoreType.DMA] * 6,
    )(body)


def _sc_gather(t, row2, col2):
    return _sc_gather_kernel(EH, EWH, NCH2, CH2)(t, row2, col2)


@functools.lru_cache(maxsize=None)
def _sc_scatter_kernel(ew, nch, ch):
    def body(he_h, col_h, zero_h, out_h, idx, buf0, buf1, acc, l0, l1, s0, s1):
        c = lax.axis_index("c")
        s = lax.axis_index("s")
        w = c * NS + s

        @pl.when(s == 0)
        def _():
            pltpu.sync_copy(zero_h, acc)

        pltpu.sync_copy(col_h.at[w], idx)

        def load(j, buf, ls, ss):
            @pl.when(j >= 2)
            def _():
                pltpu.make_async_copy(buf, acc.at[idx.at[0]], ss).wait()

            pltpu.async_copy(he_h.at[pl.ds(w * ew + j * ch, ch), :], buf, ls)

        def process(j, buf, ls, ss):
            pltpu.make_async_copy(he_h.at[pl.ds(0, ch), :], buf, ls).wait()
            pltpu.make_async_copy(buf, acc.at[idx.at[j]], ss).start(add=True)

        load(0, buf0, l0, s0)
        plsc.subcore_barrier()

        def loop(k, carry):
            j0 = 2 * k
            load(j0 + 1, buf1, l1, s1)
            process(j0, buf0, l0, s0)

            @pl.when(j0 + 2 < nch)
            def _():
                load(j0 + 2, buf0, l0, s0)

            process(j0 + 1, buf1, l1, s1)
            return carry

        lax.fori_loop(0, nch // 2, loop, 0)
        process(nch - 1, buf0, l0, s0)
        pltpu.make_async_copy(buf0, acc.at[idx.at[0]], s0).wait()
        pltpu.make_async_copy(buf1, acc.at[idx.at[0]], s1).wait()
        plsc.subcore_barrier()
        pltpu.sync_copy(acc.at[pl.ds(s * NPT, NPT), :], out_h.at[c, s])

    return functools.partial(
        pl.kernel,
        out_type=jax.ShapeDtypeStruct((NC, NS, NPT, 2 * H), _f32),
        mesh=_mesh(),
        scratch_types=[
            pltpu.VMEM((nch, ch), jnp.int32),
            pltpu.VMEM((ch, 2 * H), _f32),
            pltpu.VMEM((ch, 2 * H), _f32),
            pltpu.VMEM_SHARED((NPAD, 2 * H), _f32),
        ] + [pltpu.SemaphoreType.DMA] * 4,
    )(body)


def _sc_scatter(he, col2, zeros_nh):
    return _sc_scatter_kernel(EWH, NCH2, CH2)(he, col2, zeros_nh)


@functools.lru_cache(maxsize=None)
def _sc_count_kernel():
    return functools.partial(
        pl.kernel,
        out_type=jax.ShapeDtypeStruct((NC, NS, NPT, 128), _f32),
        mesh=_mesh(),
        scratch_types=[
            pltpu.VMEM((NCH, CH), jnp.int32),
            pltpu.VMEM((CH, 128), _f32),
            pltpu.VMEM_SHARED((NPAD, 128), _f32),
        ],
    )(_sc_count_body)


def _sc_count(col2, ones, zeros):
    return _sc_count_kernel()(col2, ones, zeros)


def _sc_count_body(col_h, ones_h, zero_h, out_h, idx, buf, acc):
    c = lax.axis_index("c")
    s = lax.axis_index("s")
    w = c * NS + s

    @pl.when(s == 0)
    def _():
        pltpu.sync_copy(zero_h, acc)

    pltpu.sync_copy(col_h.at[w], idx)
    pltpu.sync_copy(ones_h, buf)
    plsc.subcore_barrier()

    def chunk(j, carry):
        pltpu.sync_copy(buf, acc.at[idx.at[j]], add=True)
        return carry

    lax.fori_loop(0, NCH, chunk, 0)
    plsc.subcore_barrier()
    pltpu.sync_copy(acc.at[pl.ds(s * NPT, NPT), :], out_h.at[c, s])


def kernel(x, edge_index, edge_attr, u, batch, params):
    del batch
    row3 = edge_index[0].astype(jnp.int32).reshape(2, NW, NCH2, CH2)
    col3 = edge_index[1].astype(jnp.int32).reshape(2, NW, NCH2, CH2)
    col2 = edge_index[1].astype(jnp.int32).reshape(NW, NCH, CH)

    def vec(p, k):
        return p[k].reshape(1, -1)

    pe, px, pu = params["core_e"], params["core_x"], params["core_u"]
    We, Wx, Wu = pe["W1"], px["W1"], pu["W1"]
    Wes_e, Wes_h = We[0:64], We[64:128]
    Wed_e, Wed_h = We[128:192], We[192:256]
    Wee_e, Wee_h = We[256:320], We[320:384]
    Weu_e, Weu_h = We[384:448], We[448:512]
    Wxx_e, Wxx_h = Wx[0:64], Wx[64:128]
    Wxa = Wx[128:192]
    Wxu_e, Wxu_h = Wx[192:256], Wx[256:320]
    Wu_x, Wu_e = Wu[0:64], Wu[64:128]
    Wu_ue, Wu_uh = Wu[128:192], Wu[192:256]
    b1e = vec(pe, "b1")
    b1x = vec(px, "b1")

    p = params["enc_x"]
    enc_x, T, Ax, Bx, Nx = _enc_node_call(
        x, [p["W1"], vec(p, "b1"), p["W2"], vec(p, "b2"), vec(p, "ln_g"),
            vec(p, "ln_b"), Wes_e, Wes_h, Wed_e, Wed_h, Wxx_e])
    p = params["enc_e"]
    enc_e_args = [p["W1"], vec(p, "b1"), p["W2"], vec(p, "b2"),
                  vec(p, "ln_g"), vec(p, "ln_b"), Wee_e, Wee_h]
    heP0, C0 = _enc_edge_call(edge_attr.T, enc_e_args, 0)
    heP1, C1 = _enc_edge_call(edge_attr.T, enc_e_args, 1)
    p = params["enc_u"]
    enc_u, ue, ux = _u_enc_call(
        u, [p["W1"], vec(p, "b1"), p["W2"], vec(p, "b2"), vec(p, "ln_g"),
            vec(p, "ln_b"), Weu_e, Weu_h, b1e, Wxu_e, Wxu_h, b1x])

    cnt = _sc_count(col2, jnp.ones((CH, 128), _f32),
                    jnp.zeros((NPAD, 128), _f32)).reshape(NC, NPAD, 128)
    zeros_nh = jnp.zeros((NPAD, 2 * H), _f32)

    hx, hu = enc_x, enc_u
    pd = params["dec_u"]
    u_out = None
    ec_args = [Wee_h, pe["W2"], vec(pe, "b2"), vec(pe, "ln_g"),
               vec(pe, "ln_b")]
    for _ in range(NSTEPS):
        S0 = _sc_gather(T, row3[0], col3[0])
        heP0, esum0 = _edge_core_call(S0, C0, heP0, ue, ec_args)
        S1 = _sc_gather(T, row3[1], col3[1])
        heP1, esum1 = _edge_core_call(S1, C1, heP1, ue, ec_args)
        ns0 = _sc_scatter(heP0, col3[0], zeros_nh).reshape(NC, NPAD, 2 * H)
        ns1 = _sc_scatter(heP1, col3[1], zeros_nh).reshape(NC, NPAD, 2 * H)
        hx, T, xsum = _node_core_call(
            ns0, ns1, cnt, hx, Nx, Ax, Bx, ux,
            [Wxx_h, Wxa, px["W2"], vec(px, "b2"), vec(px, "ln_g"),
             vec(px, "ln_b"), Wes_h, Wed_h])
        hu, ue, ux, u_out = _global_call(
            xsum, esum0, esum1, enc_u, hu,
            [Wu_x, Wu_e, Wu_ue, Wu_uh, vec(pu, "b1"), pu["W2"], vec(pu, "b2"),
             vec(pu, "ln_g"), vec(pu, "ln_b"), Weu_e, Weu_h, b1e, Wxu_e,
             Wxu_h, b1x, pd["W1"], vec(pd, "b1"), pd["W2"], vec(pd, "b2"),
             vec(pd, "ln_g"), vec(pd, "ln_b")])

    p = params["dec_e"]
    dec_e_args = [p["W1"], vec(p, "b1"), p["W2"], p["b2"].reshape(-1, 1),
                  p["ln_g"].reshape(-1, 1), p["ln_b"].reshape(-1, 1)]
    e_out = jnp.concatenate(
        [_dec_e_call(heP0, dec_e_args), _dec_e_call(heP1, dec_e_args)],
        axis=1).T
    p = params["dec_x"]
    x_out = _dec_call(hx, [p["W1"], vec(p, "b1"), p["W2"], vec(p, "b2"),
                           vec(p, "ln_g"), vec(p, "ln_b")], N, BN, 128)
    return (e_out, x_out, u_out)


# speedup vs baseline: 1.0363x; 1.0363x over previous
import functools

import jax
import jax.numpy as jnp
from jax import lax
from jax.experimental import pallas as pl
from jax.experimental.pallas import tpu as pltpu
from jax.experimental.pallas import tpu_sc as plsc

N = 10000
E = 320000
H = 64
NSTEPS = 3

NC, NS = 2, 16
NW = NC * NS
CH = 80
EW = E // NW
NCH = EW // CH
EH = E // 2
EWH = EH // NW
CH2 = 40
NCH2 = EWH // CH2
NPAD = 10240
NPT = NPAD // NS

BE = 4000
BT = 6400
BN = 2000

_f32 = jnp.float32


def _ln(h, g, b):
    mu = jnp.mean(h, axis=-1, keepdims=True)
    d = h - mu
    var = jnp.mean(d * d, axis=-1, keepdims=True)
    return d / jnp.sqrt(var + 1e-5) * g + b


def _dot(a, b):
    return jnp.dot(a, b, preferred_element_type=_f32)


def _full(shape):
    return pl.BlockSpec(shape, lambda i: tuple(0 for _ in shape))


def _rows(bshape):
    return pl.BlockSpec(bshape, lambda i: (i,) + tuple(0 for _ in bshape[1:]))


def _enc_node_body(x, W1, b1, W2, b2, g, b, Wse, Wsh, Wde, Wdh, Wxe,
                   o_enc, o_T0, o_Ax, o_Bx, o_Nx):
    h = jnp.maximum(_dot(x[...], W1[...]) + b1[...], 0.0)
    e = _ln(_dot(h, W2[...]) + b2[...], g[...], b[...])
    o_enc[...] = e
    ax = _dot(e, Wse[...])
    bx = _dot(e, Wde[...])
    o_Ax[...] = ax
    o_Bx[...] = bx
    o_T0[...] = jnp.concatenate(
        [ax + _dot(e, Wsh[...]), bx + _dot(e, Wdh[...])], axis=-1)
    o_Nx[...] = _dot(e, Wxe[...])


def _enc_node_call(x, args):
    outs = [jax.ShapeDtypeStruct((N, H), _f32),
            jax.ShapeDtypeStruct((N, 2 * H), _f32)] \
        + [jax.ShapeDtypeStruct((N, H), _f32)] * 3
    return pl.pallas_call(
        _enc_node_body,
        grid=(N // BN,),
        in_specs=[_rows((BN, 128))] + [_full(a.shape) for a in args],
        out_specs=[_rows((BN, H)), _rows((BN, 2 * H))] + [_rows((BN, H))] * 3,
        out_shape=outs,
    )(x, *args)


def _enc_edge_body(ea_t, W1, b1, W2, b2, g, b, Wee, Weh, o_heP, o_C):
    h = jnp.maximum(
        lax.dot_general(ea_t[...], W1[...], (((0,), (0,)), ((), ())),
                        preferred_element_type=_f32) + b1[...], 0.0)
    e = _ln(_dot(h, W2[...]) + b2[...], g[...], b[...])
    o_heP[...] = jnp.concatenate([e, _dot(e, Weh[...])], axis=-1)
    o_C[...] = _dot(e, Wee[...]).astype(jnp.bfloat16)


def _enc_edge_call(ea_t, args, half):
    off = half * (EH // BT)
    return pl.pallas_call(
        _enc_edge_body,
        grid=(EH // BT,),
        in_specs=[pl.BlockSpec((16, BT), lambda i, o=off: (0, i + o))]
        + [_full(a.shape) for a in args],
        out_specs=[_rows((BT, 2 * H)), _rows((BT, H))],
        out_shape=[jax.ShapeDtypeStruct((EH, 2 * H), _f32),
                   jax.ShapeDtypeStruct((EH, H), jnp.bfloat16)],
    )(ea_t, *args)


def _u_enc_body(u, W1, b1, W2, b2, g, b, Weu_e, Weu_h, b1e, Wxu_e, Wxu_h, b1x,
                o_encu, o_ue, o_ux):
    h = jnp.maximum(_dot(u[...], W1[...]) + b1[...], 0.0)
    eu = _ln(_dot(h, W2[...]) + b2[...], g[...], b[...])
    o_encu[...] = eu
    o_ue[...] = _dot(eu, Weu_e[...]) + _dot(eu, Weu_h[...]) + b1e[...]
    o_ux[...] = _dot(eu, Wxu_e[...]) + _dot(eu, Wxu_h[...]) + b1x[...]


def _u_enc_call(u, args):
    return pl.pallas_call(
        _u_enc_body,
        out_shape=[jax.ShapeDtypeStruct((1, H), _f32)] * 3,
    )(u, *args)


def _edge_core_body(S, C, P, ue, Weh, W2, b2, g, b, o_heP, o_esum):
    h1 = S[...].astype(_f32) + C[...].astype(_f32) + P[...][:, H:] + ue[...]
    out = _ln(_dot(jnp.maximum(h1, 0.0), W2[...]) + b2[...], g[...], b[...])
    o_heP[...] = jnp.concatenate([out, _dot(out, Weh[...])], axis=-1)

    @pl.when(pl.program_id(0) == 0)
    def _():
        o_esum[...] = jnp.zeros_like(o_esum)

    o_esum[...] += jnp.sum(out, axis=0, keepdims=True)


def _edge_core_call(S, C, heP, ue, args):
    return pl.pallas_call(
        _edge_core_body,
        grid=(EH // BE,),
        in_specs=[_rows((BE, H))] * 2
        + [_rows((BE, 2 * H)), _full((1, H))]
        + [_full(a.shape) for a in args],
        out_specs=[_rows((BE, 2 * H)), _full((1, H))],
        out_shape=[jax.ShapeDtypeStruct((EH, 2 * H), _f32),
                   jax.ShapeDtypeStruct((1, H), _f32)],
    )(S, C, heP, ue, *args)


def _node_core_body(ns, ns1, cnt, hx, Nx, Ax, Bx, ux, Wxh, Wxa, W2, b2, g, b,
                    Wsh, Wdh, o_hx, o_T, o_xsum):
    s = ns[0, :, :H] + ns[1, :, :H] + ns1[0, :, :H] + ns1[1, :, :H]
    c = cnt[0, :, 0:1] + cnt[1, :, 0:1]
    eagg = s / jnp.maximum(c, 1.0)
    h1 = Nx[...] + _dot(hx[...], Wxh[...]) + _dot(eagg, Wxa[...]) + ux[...]
    hxn = _ln(_dot(jnp.maximum(h1, 0.0), W2[...]) + b2[...], g[...], b[...])
    o_hx[...] = hxn
    o_T[...] = jnp.concatenate(
        [Ax[...] + _dot(hxn, Wsh[...]), Bx[...] + _dot(hxn, Wdh[...])],
        axis=-1)

    @pl.when(pl.program_id(0) == 0)
    def _():
        o_xsum[...] = jnp.zeros_like(o_xsum)

    o_xsum[...] += jnp.sum(hxn, axis=0, keepdims=True)


def _node_core_call(ns, ns1, cnt, hx, Nx, Ax, Bx, ux, args):
    return pl.pallas_call(
        _node_core_body,
        grid=(N // BN,),
        in_specs=[pl.BlockSpec((2, BN, 2 * H), lambda i: (0, i, 0)),
                  pl.BlockSpec((2, BN, 2 * H), lambda i: (0, i, 0)),
                  pl.BlockSpec((2, BN, 128), lambda i: (0, i, 0))]
        + [_rows((BN, H))] * 4 + [_full((1, H))]
        + [_full(a.shape) for a in args],
        out_specs=[_rows((BN, H)), _rows((BN, 2 * H)), _full((1, H))],
        out_shape=[jax.ShapeDtypeStruct((N, H), _f32),
                   jax.ShapeDtypeStruct((N, 2 * H), _f32),
                   jax.ShapeDtypeStruct((1, H), _f32)],
    )(ns, ns1, cnt, hx, Nx, Ax, Bx, ux, *args)


def _global_body(xsum, esum, esum1, encu, hu, Wu_x, Wu_e, Wu_ue, Wu_uh, b1u,
                 W2u, b2u, gu, bu, Weu_e, Weu_h, b1e, Wxu_e, Wxu_h, b1x,
                 Wd1, bd1, Wd2, bd2, gd, bd,
                 o_hu, o_ue, o_ux, o_uout):
    xa = xsum[...] * (1.0 / N)
    ea = (esum[...] + esum1[...]) * (1.0 / E)
    h1 = (_dot(xa, Wu_x[...]) + _dot(ea, Wu_e[...]) + _dot(encu[...], Wu_ue[...])
          + _dot(hu[...], Wu_uh[...]) + b1u[...])
    hun = _ln(_dot(jnp.maximum(h1, 0.0), W2u[...]) + b2u[...], gu[...], bu[...])
    o_hu[...] = hun
    o_ue[...] = _dot(encu[...], Weu_e[...]) + _dot(hun, Weu_h[...]) + b1e[...]
    o_ux[...] = _dot(encu[...], Wxu_e[...]) + _dot(hun, Wxu_h[...]) + b1x[...]
    hd = jnp.maximum(_dot(hun, Wd1[...]) + bd1[...], 0.0)
    o_uout[...] = _ln(_dot(hd, Wd2[...]) + bd2[...], gd[...], bd[...])


def _global_call(xsum, esum, esum1, encu, hu, args):
    return pl.pallas_call(
        _global_body,
        out_shape=[jax.ShapeDtypeStruct((1, H), _f32)] * 3
        + [jax.ShapeDtypeStruct((1, 16), _f32)],
    )(xsum, esum, esum1, encu, hu, *args)


def _dec_body(h, W1, b1, W2, b2, g, b, o):
    a = jnp.maximum(_dot(h[...][:, :H], W1[...]) + b1[...], 0.0)
    o[...] = _ln(_dot(a, W2[...]) + b2[...], g[...], b[...])


def _dec_e_body(h, W1, b1, W2, b2, g, b, o):
    a = jnp.maximum(_dot(h[...][:, :H], W1[...]) + b1[...], 0.0)
    ot = lax.dot_general(W2[...], a, (((0,), (1,)), ((), ())),
                         preferred_element_type=_f32) + b2[...]
    mu = jnp.mean(ot, axis=0, keepdims=True)
    d = ot - mu
    var = jnp.mean(d * d, axis=0, keepdims=True)
    o[...] = d / jnp.sqrt(var + 1e-5) * g[...] + b[...]


def _dec_e_call(h, args):
    return pl.pallas_call(
        _dec_e_body,
        grid=(EH // BT,),
        in_specs=[_rows((BT, 2 * H))] + [_full(a.shape) for a in args],
        out_specs=pl.BlockSpec((16, BT), lambda i: (0, i)),
        out_shape=jax.ShapeDtypeStruct((16, EH), _f32),
    )(h, *args)


def _dec_call(h, args, rows, brows, fout):
    return pl.pallas_call(
        _dec_body,
        grid=(rows // brows,),
        in_specs=[_rows((brows, h.shape[1]))] + [_full(a.shape) for a in args],
        out_specs=_rows((brows, fout)),
        out_shape=jax.ShapeDtypeStruct((rows, fout), _f32),
    )(h, *args)


def _mesh():
    return plsc.VectorSubcoreMesh(core_axis_name="c", subcore_axis_name="s",
                                  num_cores=NC, num_subcores=NS)


@functools.lru_cache(maxsize=None)
def _sc_gather_kernel(etot, ew, nch, ch):
    def body(t_h, row_h, col_h, s_h, idxa, idxb, ba0, bb0, ba1, bb1,
             sb0, sb1, ga0, gb0, ga1, gb1, w0, w1):
        c = lax.axis_index("c")
        s = lax.axis_index("s")
        w = c * NS + s
        pltpu.sync_copy(row_h.at[w], idxa)
        pltpu.sync_copy(col_h.at[w], idxb)

        def start(j, bufa, bufb, sa, sb):
            pltpu.async_copy(t_h.at[idxa.at[j]], bufa, sa)
            pltpu.async_copy(t_h.at[idxb.at[j]], bufb, sb)

        def finish(j, bufa, bufb, sa, sb, sbuf, ws):
            pltpu.make_async_copy(t_h.at[pl.ds(0, ch)], bufa, sa).wait()
            pltpu.make_async_copy(t_h.at[pl.ds(0, ch)], bufb, sb).wait()

            @pl.when(j >= 2)
            def _():
                pltpu.make_async_copy(sbuf, s_h.at[pl.ds(0, ch), :], ws).wait()

            def add(r, carry):
                for l in range(H // 16):
                    va = bufa[r, pl.ds(l * 16, 16)]
                    vb = bufb[r, pl.ds(H + l * 16, 16)]
                    sbuf[r, pl.ds(l * 16, 16)] = va + vb
                return carry

            lax.fori_loop(0, ch, add, 0)
            pltpu.async_copy(sbuf, s_h.at[pl.ds(w * ew + j * ch, ch), :], ws)

        start(0, ba0, bb0, ga0, gb0)

        def loop(k, carry):
            j0 = 2 * k
            start(j0 + 1, ba1, bb1, ga1, gb1)
            finish(j0, ba0, bb0, ga0, gb0, sb0, w0)

            @pl.when(j0 + 2 < nch)
            def _():
                start(j0 + 2, ba0, bb0, ga0, gb0)

            finish(j0 + 1, ba1, bb1, ga1, gb1, sb1, w1)
            return carry

        lax.fori_loop(0, nch // 2, loop, 0)
        finish(nch - 1, ba0, bb0, ga0, gb0, sb0, w0)
        pltpu.make_async_copy(sb0, s_h.at[pl.ds(0, ch), :], w0).wait()
        pltpu.make_async_copy(sb1, s_h.at[pl.ds(0, ch), :], w1).wait()

    return functools.partial(
        pl.kernel,
        out_type=jax.ShapeDtypeStruct((etot, H), _f32),
        mesh=_mesh(),
        scratch_types=[
            pltpu.VMEM((nch, ch), jnp.int32),
            pltpu.VMEM((nch, ch), jnp.int32),
            pltpu.VMEM((ch, 2 * H), _f32),
            pltpu.VMEM((ch, 2 * H), _f32),
            pltpu.VMEM((ch, 2 * H), _f32),
            pltpu.VMEM((ch, 2 * H), _f32),
            pltpu.VMEM((ch, H), _f32),
            pltpu.VMEM((ch, H), _f32),
        ] + [pltpu.SemaphoreType.DMA] * 6,
    )(body)


def _sc_gather(t, row2, col2):
    return _sc_gather_kernel(EH, EWH, NCH2, CH2)(t, row2, col2)


@functools.lru_cache(maxsize=None)
def _sc_scatter_kernel(ew, nch, ch):
    def body(he_h, col_h, zero_h, out_h, idx, buf0, buf1, acc, l0, l1, s0, s1):
        c = lax.axis_index("c")
        s = lax.axis_index("s")
        w = c * NS + s

        @pl.when(s == 0)
        def _():
            pltpu.sync_copy(zero_h, acc)

        pltpu.sync_copy(col_h.at[w], idx)

        def load(j, buf, ls, ss):
            @pl.when(j >= 2)
            def _():
                pltpu.make_async_copy(buf, acc.at[idx.at[0]], ss).wait()

            pltpu.async_copy(he_h.at[pl.ds(w * ew + j * ch, ch), :], buf, ls)

        def process(j, buf, ls, ss):
            pltpu.make_async_copy(he_h.at[pl.ds(0, ch), :], buf, ls).wait()
            pltpu.make_async_copy(buf, acc.at[idx.at[j]], ss).start(add=True)

        load(0, buf0, l0, s0)
        plsc.subcore_barrier()

        def loop(k, carry):
            j0 = 2 * k
            load(j0 + 1, buf1, l1, s1)
            process(j0, buf0, l0, s0)

            @pl.when(j0 + 2 < nch)
            def _():
                load(j0 + 2, buf0, l0, s0)

            process(j0 + 1, buf1, l1, s1)
            return carry

        lax.fori_loop(0, nch // 2, loop, 0)
        process(nch - 1, buf0, l0, s0)
        pltpu.make_async_copy(buf0, acc.at[idx.at[0]], s0).wait()
        pltpu.make_async_copy(buf1, acc.at[idx.at[0]], s1).wait()
        plsc.subcore_barrier()
        pltpu.sync_copy(acc.at[pl.ds(s * NPT, NPT), :], out_h.at[c, s])

    return functools.partial(
        pl.kernel,
        out_type=jax.ShapeDtypeStruct((NC, NS, NPT, 2 * H), _f32),
        mesh=_mesh(),
        scratch_types=[
            pltpu.VMEM((nch, ch), jnp.int32),
            pltpu.VMEM((ch, 2 * H), _f32),
            pltpu.VMEM((ch, 2 * H), _f32),
            pltpu.VMEM_SHARED((NPAD, 2 * H), _f32),
        ] + [pltpu.SemaphoreType.DMA] * 4,
    )(body)


def _sc_scatter(he, col2, zeros_nh):
    return _sc_scatter_kernel(EWH, NCH2, CH2)(he, col2, zeros_nh)


@functools.lru_cache(maxsize=None)
def _sc_count_kernel():
    return functools.partial(
        pl.kernel,
        out_type=jax.ShapeDtypeStruct((NC, NS, NPT, 128), _f32),
        mesh=_mesh(),
        scratch_types=[
            pltpu.VMEM((NCH, CH), jnp.int32),
            pltpu.VMEM((CH, 128), _f32),
            pltpu.VMEM_SHARED((NPAD, 128), _f32),
        ],
    )(_sc_count_body)


def _sc_count(col2, ones, zeros):
    return _sc_count_kernel()(col2, ones, zeros)


def _sc_count_body(col_h, ones_h, zero_h, out_h, idx, buf, acc):
    c = lax.axis_index("c")
    s = lax.axis_index("s")
    w = c * NS + s

    @pl.when(s == 0)
    def _():
        pltpu.sync_copy(zero_h, acc)

    pltpu.sync_copy(col_h.at[w], idx)
    pltpu.sync_copy(ones_h, buf)
    plsc.subcore_barrier()

    def chunk(j, carry):
        pltpu.sync_copy(buf, acc.at[idx.at[j]], add=True)
        return carry

    lax.fori_loop(0, NCH, chunk, 0)
    plsc.subcore_barrier()
    pltpu.sync_copy(acc.at[pl.ds(s * NPT, NPT), :], out_h.at[c, s])


def kernel(x, edge_index, edge_attr, u, batch, params):
    del batch
    row3 = edge_index[0].astype(jnp.int32).reshape(2, NW, NCH2, CH2)
    col3 = edge_index[1].astype(jnp.int32).reshape(2, NW, NCH2, CH2)
    col2 = edge_index[1].astype(jnp.int32).reshape(NW, NCH, CH)

    def vec(p, k):
        return p[k].reshape(1, -1)

    pe, px, pu = params["core_e"], params["core_x"], params["core_u"]
    We, Wx, Wu = pe["W1"], px["W1"], pu["W1"]
    Wes_e, Wes_h = We[0:64], We[64:128]
    Wed_e, Wed_h = We[128:192], We[192:256]
    Wee_e, Wee_h = We[256:320], We[320:384]
    Weu_e, Weu_h = We[384:448], We[448:512]
    Wxx_e, Wxx_h = Wx[0:64], Wx[64:128]
    Wxa = Wx[128:192]
    Wxu_e, Wxu_h = Wx[192:256], Wx[256:320]
    Wu_x, Wu_e = Wu[0:64], Wu[64:128]
    Wu_ue, Wu_uh = Wu[128:192], Wu[192:256]
    b1e = vec(pe, "b1")
    b1x = vec(px, "b1")

    p = params["enc_x"]
    enc_x, T, Ax, Bx, Nx = _enc_node_call(
        x, [p["W1"], vec(p, "b1"), p["W2"], vec(p, "b2"), vec(p, "ln_g"),
            vec(p, "ln_b"), Wes_e, Wes_h, Wed_e, Wed_h, Wxx_e])
    p = params["enc_e"]
    enc_e_args = [p["W1"], vec(p, "b1"), p["W2"], vec(p, "b2"),
                  vec(p, "ln_g"), vec(p, "ln_b"), Wee_e, Wee_h]
    heP0, C0 = _enc_edge_call(edge_attr.T, enc_e_args, 0)
    heP1, C1 = _enc_edge_call(edge_attr.T, enc_e_args, 1)
    p = params["enc_u"]
    enc_u, ue, ux = _u_enc_call(
        u, [p["W1"], vec(p, "b1"), p["W2"], vec(p, "b2"), vec(p, "ln_g"),
            vec(p, "ln_b"), Weu_e, Weu_h, b1e, Wxu_e, Wxu_h, b1x])

    cnt = _sc_count(col2, jnp.ones((CH, 128), _f32),
                    jnp.zeros((NPAD, 128), _f32)).reshape(NC, NPAD, 128)
    zeros_nh = jnp.zeros((NPAD, 2 * H), _f32)

    hx, hu = enc_x, enc_u
    pd = params["dec_u"]
    u_out = None
    ec_args = [Wee_h, pe["W2"], vec(pe, "b2"), vec(pe, "ln_g"),
               vec(pe, "ln_b")]
    for _ in range(NSTEPS):
        S0 = _sc_gather(T, row3[0], col3[0])
        heP0, esum0 = _edge_core_call(S0, C0, heP0, ue, ec_args)
        S1 = _sc_gather(T, row3[1], col3[1])
        heP1, esum1 = _edge_core_call(S1, C1, heP1, ue, ec_args)
        ns0 = _sc_scatter(heP0, col3[0], zeros_nh).reshape(NC, NPAD, 2 * H)
        ns1 = _sc_scatter(heP1, col3[1], zeros_nh).reshape(NC, NPAD, 2 * H)
        hx, T, xsum = _node_core_call(
            ns0, ns1, cnt, hx, Nx, Ax, Bx, ux,
            [Wxx_h, Wxa, px["W2"], vec(px, "b2"), vec(px, "ln_g"),
             vec(px, "ln_b"), Wes_h, Wed_h])
        hu, ue, ux, u_out = _global_call(
            xsum, esum0, esum1, enc_u, hu,
            [Wu_x, Wu_e, Wu_ue, Wu_uh, vec(pu, "b1"), pu["W2"], vec(pu, "b2"),
             vec(pu, "ln_g"), vec(pu, "ln_b"), Weu_e, Weu_h, b1e, Wxu_e,
             Wxu_h, b1x, pd["W1"], vec(pd, "b1"), pd["W2"], vec(pd, "b2"),
             vec(pd, "ln_g"), vec(pd, "ln_b")])

    p = params["dec_e"]
    dec_e_args = [p["W1"], vec(p, "b1"), p["W2"], p["b2"].reshape(-1, 1),
                  p["ln_g"].reshape(-1, 1), p["ln_b"].reshape(-1, 1)]
    e_out = jnp.concatenate(
        [_dec_e_call(heP0, dec_e_args), _dec_e_call(heP1, dec_e_args)],
        axis=1).T
    p = params["dec_x"]
    x_out = _dec_call(hx, [p["W1"], vec(p, "b1"), p["W2"], vec(p, "b2"),
                           vec(p, "ln_g"), vec(p, "ln_b")], N, BN, 128)
    return (e_out, x_out, u_out)


# speedup vs baseline: 1.0367x; 1.0004x over previous
import functools

import jax
import jax.numpy as jnp
from jax import lax
from jax.experimental import pallas as pl
from jax.experimental.pallas import tpu as pltpu
from jax.experimental.pallas import tpu_sc as plsc

N = 10000
E = 320000
H = 64
NSTEPS = 3

NC, NS = 2, 16
NW = NC * NS
CH = 80
EW = E // NW
NCH = EW // CH
EH = E // 2
EWH = EH // NW
CH2 = 40
NCH2 = EWH // CH2
NPAD = 10240
NPT = NPAD // NS

BE = 4000
BT = 6400
BN = 2000

_f32 = jnp.float32


def _ln(h, g, b):
    mu = jnp.mean(h, axis=-1, keepdims=True)
    d = h - mu
    var = jnp.mean(d * d, axis=-1, keepdims=True)
    return d / jnp.sqrt(var + 1e-5) * g + b


def _dot(a, b):
    return jnp.dot(a, b, preferred_element_type=_f32)


def _full(shape):
    return pl.BlockSpec(shape, lambda i: tuple(0 for _ in shape))


def _rows(bshape):
    return pl.BlockSpec(bshape, lambda i: (i,) + tuple(0 for _ in bshape[1:]))


def _enc_node_body(x, W1, b1, W2, b2, g, b, Wse, Wsh, Wde, Wdh, Wxe,
                   o_enc, o_T0, o_Ax, o_Bx, o_Nx):
    h = jnp.maximum(_dot(x[...], W1[...]) + b1[...], 0.0)
    e = _ln(_dot(h, W2[...]) + b2[...], g[...], b[...])
    o_enc[...] = e
    ax = _dot(e, Wse[...])
    bx = _dot(e, Wde[...])
    o_Ax[...] = ax
    o_Bx[...] = bx
    o_T0[...] = jnp.concatenate(
        [ax + _dot(e, Wsh[...]), bx + _dot(e, Wdh[...])], axis=-1)
    o_Nx[...] = _dot(e, Wxe[...])


def _enc_node_call(x, args):
    outs = [jax.ShapeDtypeStruct((N, H), _f32),
            jax.ShapeDtypeStruct((N, 2 * H), _f32)] \
        + [jax.ShapeDtypeStruct((N, H), _f32)] * 3
    return pl.pallas_call(
        _enc_node_body,
        grid=(N // BN,),
        in_specs=[_rows((BN, 128))] + [_full(a.shape) for a in args],
        out_specs=[_rows((BN, H)), _rows((BN, 2 * H))] + [_rows((BN, H))] * 3,
        out_shape=outs,
    )(x, *args)


def _enc_edge_body(ea_t, W1, b1, W2, b2, g, b, Wee, Weh, o_heP, o_C):
    h = jnp.maximum(
        lax.dot_general(ea_t[...], W1[...], (((0,), (0,)), ((), ())),
                        preferred_element_type=_f32) + b1[...], 0.0)
    e = _ln(_dot(h, W2[...]) + b2[...], g[...], b[...])
    o_heP[...] = jnp.concatenate([e, _dot(e, Weh[...])], axis=-1)
    o_C[...] = _dot(e, Wee[...]).astype(jnp.bfloat16)


def _enc_edge_call(ea_t, args, half):
    off = half * (EH // BT)
    return pl.pallas_call(
        _enc_edge_body,
        grid=(EH // BT,),
        in_specs=[pl.BlockSpec((16, BT), lambda i, o=off: (0, i + o))]
        + [_full(a.shape) for a in args],
        out_specs=[_rows((BT, 2 * H)), _rows((BT, H))],
        out_shape=[jax.ShapeDtypeStruct((EH, 2 * H), _f32),
                   jax.ShapeDtypeStruct((EH, H), jnp.bfloat16)],
    )(ea_t, *args)


def _u_enc_body(u, W1, b1, W2, b2, g, b, Weu_e, Weu_h, b1e, Wxu_e, Wxu_h, b1x,
                o_encu, o_ue, o_ux):
    h = jnp.maximum(_dot(u[...], W1[...]) + b1[...], 0.0)
    eu = _ln(_dot(h, W2[...]) + b2[...], g[...], b[...])
    o_encu[...] = eu
    o_ue[...] = _dot(eu, Weu_e[...]) + _dot(eu, Weu_h[...]) + b1e[...]
    o_ux[...] = _dot(eu, Wxu_e[...]) + _dot(eu, Wxu_h[...]) + b1x[...]


def _u_enc_call(u, args):
    return pl.pallas_call(
        _u_enc_body,
        out_shape=[jax.ShapeDtypeStruct((1, H), _f32)] * 3,
    )(u, *args)


def _edge_core_body(S, C, P, ue, Weh, W2, b2, g, b, o_heP, o_esum):
    h1 = S[...] + C[...].astype(_f32) + P[...][:, H:] + ue[...]
    out = _ln(_dot(jnp.maximum(h1, 0.0), W2[...]) + b2[...], g[...], b[...])
    o_heP[...] = jnp.concatenate([out, _dot(out, Weh[...])], axis=-1)

    @pl.when(pl.program_id(0) == 0)
    def _():
        o_esum[...] = jnp.zeros_like(o_esum)

    o_esum[...] += jnp.sum(out, axis=0, keepdims=True)


def _edge_core_call(S, C, heP, ue, args):
    return pl.pallas_call(
        _edge_core_body,
        grid=(EH // BE,),
        in_specs=[_rows((BE, H)), _rows((BE, H))]
        + [_rows((BE, 2 * H)), _full((1, H))]
        + [_full(a.shape) for a in args],
        out_specs=[_rows((BE, 2 * H)), _full((1, H))],
        out_shape=[jax.ShapeDtypeStruct((EH, 2 * H), _f32),
                   jax.ShapeDtypeStruct((1, H), _f32)],
    )(S, C, heP, ue, *args)


def _node_core_body(ns, ns1, cnt, hx, Nx, Ax, Bx, ux, Wxh, Wxa, W2, b2, g, b,
                    Wsh, Wdh, o_hx, o_T, o_xsum):
    s = ns[0, :, :H] + ns[1, :, :H] + ns1[0, :, :H] + ns1[1, :, :H]
    c = cnt[0, :, 0:1] + cnt[1, :, 0:1]
    eagg = s / jnp.maximum(c, 1.0)
    h1 = Nx[...] + _dot(hx[...], Wxh[...]) + _dot(eagg, Wxa[...]) + ux[...]
    hxn = _ln(_dot(jnp.maximum(h1, 0.0), W2[...]) + b2[...], g[...], b[...])
    o_hx[...] = hxn
    o_T[...] = jnp.concatenate(
        [Ax[...] + _dot(hxn, Wsh[...]), Bx[...] + _dot(hxn, Wdh[...])],
        axis=-1)

    @pl.when(pl.program_id(0) == 0)
    def _():
        o_xsum[...] = jnp.zeros_like(o_xsum)

    o_xsum[...] += jnp.sum(hxn, axis=0, keepdims=True)


def _node_core_call(ns, ns1, cnt, hx, Nx, Ax, Bx, ux, args):
    return pl.pallas_call(
        _node_core_body,
        grid=(N // BN,),
        in_specs=[pl.BlockSpec((2, BN, 2 * H), lambda i: (0, i, 0)),
                  pl.BlockSpec((2, BN, 2 * H), lambda i: (0, i, 0)),
                  pl.BlockSpec((2, BN, 128), lambda i: (0, i, 0))]
        + [_rows((BN, H))] * 4 + [_full((1, H))]
        + [_full(a.shape) for a in args],
        out_specs=[_rows((BN, H)), _rows((BN, 2 * H)), _full((1, H))],
        out_shape=[jax.ShapeDtypeStruct((N, H), _f32),
                   jax.ShapeDtypeStruct((N, 2 * H), _f32),
                   jax.ShapeDtypeStruct((1, H), _f32)],
    )(ns, ns1, cnt, hx, Nx, Ax, Bx, ux, *args)


def _global_body(xsum, esum, esum1, encu, hu, Wu_x, Wu_e, Wu_ue, Wu_uh, b1u,
                 W2u, b2u, gu, bu, Weu_e, Weu_h, b1e, Wxu_e, Wxu_h, b1x,
                 Wd1, bd1, Wd2, bd2, gd, bd,
                 o_hu, o_ue, o_ux, o_uout):
    xa = xsum[...] * (1.0 / N)
    ea = (esum[...] + esum1[...]) * (1.0 / E)
    h1 = (_dot(xa, Wu_x[...]) + _dot(ea, Wu_e[...]) + _dot(encu[...], Wu_ue[...])
          + _dot(hu[...], Wu_uh[...]) + b1u[...])
    hun = _ln(_dot(jnp.maximum(h1, 0.0), W2u[...]) + b2u[...], gu[...], bu[...])
    o_hu[...] = hun
    o_ue[...] = _dot(encu[...], Weu_e[...]) + _dot(hun, Weu_h[...]) + b1e[...]
    o_ux[...] = _dot(encu[...], Wxu_e[...]) + _dot(hun, Wxu_h[...]) + b1x[...]
    hd = jnp.maximum(_dot(hun, Wd1[...]) + bd1[...], 0.0)
    o_uout[...] = _ln(_dot(hd, Wd2[...]) + bd2[...], gd[...], bd[...])


def _global_call(xsum, esum, esum1, encu, hu, args):
    return pl.pallas_call(
        _global_body,
        out_shape=[jax.ShapeDtypeStruct((1, H), _f32)] * 3
        + [jax.ShapeDtypeStruct((1, 16), _f32)],
    )(xsum, esum, esum1, encu, hu, *args)


def _dec_body(h, W1, b1, W2, b2, g, b, o):
    a = jnp.maximum(_dot(h[...][:, :H], W1[...]) + b1[...], 0.0)
    o[...] = _ln(_dot(a, W2[...]) + b2[...], g[...], b[...])


def _dec_e_body(h, W1, b1, W2, b2, g, b, o):
    a = jnp.maximum(_dot(h[...][:, :H], W1[...]) + b1[...], 0.0)
    ot = lax.dot_general(W2[...], a, (((0,), (1,)), ((), ())),
                         preferred_element_type=_f32) + b2[...]
    mu = jnp.mean(ot, axis=0, keepdims=True)
    d = ot - mu
    var = jnp.mean(d * d, axis=0, keepdims=True)
    o[...] = d / jnp.sqrt(var + 1e-5) * g[...] + b[...]


def _dec_e_call(h, args):
    return pl.pallas_call(
        _dec_e_body,
        grid=(EH // BT,),
        in_specs=[_rows((BT, 2 * H))] + [_full(a.shape) for a in args],
        out_specs=pl.BlockSpec((16, BT), lambda i: (0, i)),
        out_shape=jax.ShapeDtypeStruct((16, EH), _f32),
    )(h, *args)


def _dec_call(h, args, rows, brows, fout):
    return pl.pallas_call(
        _dec_body,
        grid=(rows // brows,),
        in_specs=[_rows((brows, h.shape[1]))] + [_full(a.shape) for a in args],
        out_specs=_rows((brows, fout)),
        out_shape=jax.ShapeDtypeStruct((rows, fout), _f32),
    )(h, *args)


def _mesh():
    return plsc.VectorSubcoreMesh(core_axis_name="c", subcore_axis_name="s",
                                  num_cores=NC, num_subcores=NS)


@functools.lru_cache(maxsize=None)
def _sc_gather_kernel(etot, ew, nch, ch):
    def body(t_h, row_h, col_h, s_h, idxa, idxb, ba0, bb0, ba1, bb1,
             sb0, sb1, ga0, gb0, ga1, gb1, w0, w1):
        c = lax.axis_index("c")
        s = lax.axis_index("s")
        w = c * NS + s
        pltpu.sync_copy(row_h.at[w], idxa)
        pltpu.sync_copy(col_h.at[w], idxb)

        def start(j, bufa, bufb, sa, sb):
            pltpu.async_copy(t_h.at[idxa.at[j]], bufa, sa)
            pltpu.async_copy(t_h.at[idxb.at[j]], bufb, sb)

        def finish(j, bufa, bufb, sa, sb, sbuf, ws):
            pltpu.make_async_copy(t_h.at[pl.ds(0, ch)], bufa, sa).wait()
            pltpu.make_async_copy(t_h.at[pl.ds(0, ch)], bufb, sb).wait()

            @pl.when(j >= 2)
            def _():
                pltpu.make_async_copy(sbuf, s_h.at[pl.ds(0, ch), :], ws).wait()

            def add(r, carry):
                for l in range(H // 16):
                    va = bufa[r, pl.ds(l * 16, 16)]
                    vb = bufb[r, pl.ds(H + l * 16, 16)]
                    sbuf[r, pl.ds(l * 16, 16)] = va + vb
                return carry

            lax.fori_loop(0, ch, add, 0)
            pltpu.async_copy(sbuf, s_h.at[pl.ds(w * ew + j * ch, ch), :], ws)

        start(0, ba0, bb0, ga0, gb0)

        def loop(k, carry):
            j0 = 2 * k
            start(j0 + 1, ba1, bb1, ga1, gb1)
            finish(j0, ba0, bb0, ga0, gb0, sb0, w0)

            @pl.when(j0 + 2 < nch)
            def _():
                start(j0 + 2, ba0, bb0, ga0, gb0)

            finish(j0 + 1, ba1, bb1, ga1, gb1, sb1, w1)
            return carry

        lax.fori_loop(0, nch // 2, loop, 0)
        finish(nch - 1, ba0, bb0, ga0, gb0, sb0, w0)
        pltpu.make_async_copy(sb0, s_h.at[pl.ds(0, ch), :], w0).wait()
        pltpu.make_async_copy(sb1, s_h.at[pl.ds(0, ch), :], w1).wait()

    return functools.partial(
        pl.kernel,
        out_type=jax.ShapeDtypeStruct((etot, H), _f32),
        mesh=_mesh(),
        scratch_types=[
            pltpu.VMEM((nch, ch), jnp.int32),
            pltpu.VMEM((nch, ch), jnp.int32),
            pltpu.VMEM((ch, 2 * H), _f32),
            pltpu.VMEM((ch, 2 * H), _f32),
            pltpu.VMEM((ch, 2 * H), _f32),
            pltpu.VMEM((ch, 2 * H), _f32),
            pltpu.VMEM((ch, H), _f32),
            pltpu.VMEM((ch, H), _f32),
        ] + [pltpu.SemaphoreType.DMA] * 6,
    )(body)


def _sc_gather(t, row2, col2):
    return _sc_gather_kernel(EH, EWH, NCH2, CH2)(t, row2, col2)


@functools.lru_cache(maxsize=None)
def _sc_scatter_kernel(ew, nch, ch):
    def body(he_h, col_h, zero_h, out_h, idx, buf0, buf1, acc, l0, l1, s0, s1):
        c = lax.axis_index("c")
        s = lax.axis_index("s")
        w = c * NS + s

        @pl.when(s == 0)
        def _():
            pltpu.sync_copy(zero_h, acc)

        pltpu.sync_copy(col_h.at[w], idx)

        def load(j, buf, ls, ss):
            @pl.when(j >= 2)
            def _():
                pltpu.make_async_copy(buf, acc.at[idx.at[0]], ss).wait()

            pltpu.async_copy(he_h.at[pl.ds(w * ew + j * ch, ch), :], buf, ls)

        def process(j, buf, ls, ss):
            pltpu.make_async_copy(he_h.at[pl.ds(0, ch), :], buf, ls).wait()
            pltpu.make_async_copy(buf, acc.at[idx.at[j]], ss).start(add=True)

        load(0, buf0, l0, s0)
        plsc.subcore_barrier()

        def loop(k, carry):
            j0 = 2 * k
            load(j0 + 1, buf1, l1, s1)
            process(j0, buf0, l0, s0)

            @pl.when(j0 + 2 < nch)
            def _():
                load(j0 + 2, buf0, l0, s0)

            process(j0 + 1, buf1, l1, s1)
            return carry

        lax.fori_loop(0, nch // 2, loop, 0)
        process(nch - 1, buf0, l0, s0)
        pltpu.make_async_copy(buf0, acc.at[idx.at[0]], s0).wait()
        pltpu.make_async_copy(buf1, acc.at[idx.at[0]], s1).wait()
        plsc.subcore_barrier()
        pltpu.sync_copy(acc.at[pl.ds(s * NPT, NPT), :], out_h.at[c, s])

    return functools.partial(
        pl.kernel,
        out_type=jax.ShapeDtypeStruct((NC, NS, NPT, 2 * H), _f32),
        mesh=_mesh(),
        scratch_types=[
            pltpu.VMEM((nch, ch), jnp.int32),
            pltpu.VMEM((ch, 2 * H), _f32),
            pltpu.VMEM((ch, 2 * H), _f32),
            pltpu.VMEM_SHARED((NPAD, 2 * H), _f32),
        ] + [pltpu.SemaphoreType.DMA] * 4,
    )(body)


def _sc_scatter(he, col2, zeros_nh):
    return _sc_scatter_kernel(EWH, NCH2, CH2)(he, col2, zeros_nh)


@functools.lru_cache(maxsize=None)
def _sc_count_kernel():
    return functools.partial(
        pl.kernel,
        out_type=jax.ShapeDtypeStruct((NC, NS, NPT, 128), _f32),
        mesh=_mesh(),
        scratch_types=[
            pltpu.VMEM((NCH, CH), jnp.int32),
            pltpu.VMEM((CH, 128), _f32),
            pltpu.VMEM_SHARED((NPAD, 128), _f32),
        ],
    )(_sc_count_body)


def _sc_count(col2, ones, zeros):
    return _sc_count_kernel()(col2, ones, zeros)


def _sc_count_body(col_h, ones_h, zero_h, out_h, idx, buf, acc):
    c = lax.axis_index("c")
    s = lax.axis_index("s")
    w = c * NS + s

    @pl.when(s == 0)
    def _():
        pltpu.sync_copy(zero_h, acc)

    pltpu.sync_copy(col_h.at[w], idx)
    pltpu.sync_copy(ones_h, buf)
    plsc.subcore_barrier()

    def chunk(j, carry):
        pltpu.sync_copy(buf, acc.at[idx.at[j]], add=True)
        return carry

    lax.fori_loop(0, NCH, chunk, 0)
    plsc.subcore_barrier()
    pltpu.sync_copy(acc.at[pl.ds(s * NPT, NPT), :], out_h.at[c, s])


def kernel(x, edge_index, edge_attr, u, batch, params):
    del batch
    row3 = edge_index[0].astype(jnp.int32).reshape(2, NW, NCH2, CH2)
    col3 = edge_index[1].astype(jnp.int32).reshape(2, NW, NCH2, CH2)
    col2 = edge_index[1].astype(jnp.int32).reshape(NW, NCH, CH)

    def vec(p, k):
        return p[k].reshape(1, -1)

    pe, px, pu = params["core_e"], params["core_x"], params["core_u"]
    We, Wx, Wu = pe["W1"], px["W1"], pu["W1"]
    Wes_e, Wes_h = We[0:64], We[64:128]
    Wed_e, Wed_h = We[128:192], We[192:256]
    Wee_e, Wee_h = We[256:320], We[320:384]
    Weu_e, Weu_h = We[384:448], We[448:512]
    Wxx_e, Wxx_h = Wx[0:64], Wx[64:128]
    Wxa = Wx[128:192]
    Wxu_e, Wxu_h = Wx[192:256], Wx[256:320]
    Wu_x, Wu_e = Wu[0:64], Wu[64:128]
    Wu_ue, Wu_uh = Wu[128:192], Wu[192:256]
    b1e = vec(pe, "b1")
    b1x = vec(px, "b1")

    p = params["enc_x"]
    enc_x, T, Ax, Bx, Nx = _enc_node_call(
        x, [p["W1"], vec(p, "b1"), p["W2"], vec(p, "b2"), vec(p, "ln_g"),
            vec(p, "ln_b"), Wes_e, Wes_h, Wed_e, Wed_h, Wxx_e])
    p = params["enc_e"]
    enc_e_args = [p["W1"], vec(p, "b1"), p["W2"], vec(p, "b2"),
                  vec(p, "ln_g"), vec(p, "ln_b"), Wee_e, Wee_h]
    heP0, C0 = _enc_edge_call(edge_attr.T, enc_e_args, 0)
    heP1, C1 = _enc_edge_call(edge_attr.T, enc_e_args, 1)
    p = params["enc_u"]
    enc_u, ue, ux = _u_enc_call(
        u, [p["W1"], vec(p, "b1"), p["W2"], vec(p, "b2"), vec(p, "ln_g"),
            vec(p, "ln_b"), Weu_e, Weu_h, b1e, Wxu_e, Wxu_h, b1x])

    cnt = _sc_count(col2, jnp.ones((CH, 128), _f32),
                    jnp.zeros((NPAD, 128), _f32)).reshape(NC, NPAD, 128)
    zeros_nh = jnp.zeros((NPAD, 2 * H), _f32)

    hx, hu = enc_x, enc_u
    pd = params["dec_u"]
    u_out = None
    ec_args = [Wee_h, pe["W2"], vec(pe, "b2"), vec(pe, "ln_g"),
               vec(pe, "ln_b")]
    for _ in range(NSTEPS):
        S0 = _sc_gather(T, row3[0], col3[0])
        heP0, esum0 = _edge_core_call(S0, C0, heP0, ue, ec_args)
        S1 = _sc_gather(T, row3[1], col3[1])
        heP1, esum1 = _edge_core_call(S1, C1, heP1, ue, ec_args)
        ns0 = _sc_scatter(heP0, col3[0], zeros_nh).reshape(NC, NPAD, 2 * H)
        ns1 = _sc_scatter(heP1, col3[1], zeros_nh).reshape(NC, NPAD, 2 * H)
        hx, T, xsum = _node_core_call(
            ns0, ns1, cnt, hx, Nx, Ax, Bx, ux,
            [Wxx_h, Wxa, px["W2"], vec(px, "b2"), vec(px, "ln_g"),
             vec(px, "ln_b"), Wes_h, Wed_h])
        hu, ue, ux, u_out = _global_call(
            xsum, esum0, esum1, enc_u, hu,
            [Wu_x, Wu_e, Wu_ue, Wu_uh, vec(pu, "b1"), pu["W2"], vec(pu, "b2"),
             vec(pu, "ln_g"), vec(pu, "ln_b"), Weu_e, Weu_h, b1e, Wxu_e,
             Wxu_h, b1x, pd["W1"], vec(pd, "b1"), pd["W2"], vec(pd, "b2"),
             vec(pd, "ln_g"), vec(pd, "ln_b")])

    p = params["dec_e"]
    dec_e_args = [p["W1"], vec(p, "b1"), p["W2"], p["b2"].reshape(-1, 1),
                  p["ln_g"].reshape(-1, 1), p["ln_b"].reshape(-1, 1)]
    e_out = jnp.concatenate(
        [_dec_e_call(heP0, dec_e_args), _dec_e_call(heP1, dec_e_args)],
        axis=1).T
    p = params["dec_x"]
    x_out = _dec_call(hx, [p["W1"], vec(p, "b1"), p["W2"], vec(p, "b2"),
                           vec(p, "ln_g"), vec(p, "ln_b")], N, BN, 128)
    return (e_out, x_out, u_out)


# speedup vs baseline: 1.0375x; 1.0008x over previous
import functools

import jax
import jax.numpy as jnp
from jax import lax
from jax.experimental import pallas as pl
from jax.experimental.pallas import tpu as pltpu
from jax.experimental.pallas import tpu_sc as plsc

N = 10000
E = 320000
H = 64
NSTEPS = 3

NC, NS = 2, 16
NW = NC * NS
CH = 80
EW = E // NW
NCH = EW // CH
EH = E // 2
EWH = EH // NW
CH2 = 40
NCH2 = EWH // CH2
NPAD = 10240
NPT = NPAD // NS

BE = 8000
BT = 6400
BN = 2000

_f32 = jnp.float32


def _ln(h, g, b):
    mu = jnp.mean(h, axis=-1, keepdims=True)
    d = h - mu
    var = jnp.mean(d * d, axis=-1, keepdims=True)
    return d / jnp.sqrt(var + 1e-5) * g + b


def _dot(a, b):
    return jnp.dot(a, b, preferred_element_type=_f32)


def _full(shape):
    return pl.BlockSpec(shape, lambda i: tuple(0 for _ in shape))


def _rows(bshape):
    return pl.BlockSpec(bshape, lambda i: (i,) + tuple(0 for _ in bshape[1:]))


def _enc_node_body(x, W1, b1, W2, b2, g, b, Wse, Wsh, Wde, Wdh, Wxe,
                   o_enc, o_T0, o_Ax, o_Bx, o_Nx):
    h = jnp.maximum(_dot(x[...], W1[...]) + b1[...], 0.0)
    e = _ln(_dot(h, W2[...]) + b2[...], g[...], b[...])
    o_enc[...] = e
    ax = _dot(e, Wse[...])
    bx = _dot(e, Wde[...])
    o_Ax[...] = ax
    o_Bx[...] = bx
    o_T0[...] = jnp.concatenate(
        [ax + _dot(e, Wsh[...]), bx + _dot(e, Wdh[...])], axis=-1)
    o_Nx[...] = _dot(e, Wxe[...])


def _enc_node_call(x, args):
    outs = [jax.ShapeDtypeStruct((N, H), _f32),
            jax.ShapeDtypeStruct((N, 2 * H), _f32)] \
        + [jax.ShapeDtypeStruct((N, H), _f32)] * 3
    return pl.pallas_call(
        _enc_node_body,
        grid=(N // BN,),
        in_specs=[_rows((BN, 128))] + [_full(a.shape) for a in args],
        out_specs=[_rows((BN, H)), _rows((BN, 2 * H))] + [_rows((BN, H))] * 3,
        out_shape=outs,
    )(x, *args)


def _enc_edge_body(ea_t, W1, b1, W2, b2, g, b, Wee, Weh, o_heP, o_C):
    h = jnp.maximum(
        lax.dot_general(ea_t[...], W1[...], (((0,), (0,)), ((), ())),
                        preferred_element_type=_f32) + b1[...], 0.0)
    e = _ln(_dot(h, W2[...]) + b2[...], g[...], b[...])
    o_heP[...] = jnp.concatenate([e, _dot(e, Weh[...])], axis=-1)
    o_C[...] = _dot(e, Wee[...]).astype(jnp.bfloat16)


def _enc_edge_call(ea_t, args, half):
    off = half * (EH // BT)
    return pl.pallas_call(
        _enc_edge_body,
        grid=(EH // BT,),
        in_specs=[pl.BlockSpec((16, BT), lambda i, o=off: (0, i + o))]
        + [_full(a.shape) for a in args],
        out_specs=[_rows((BT, 2 * H)), _rows((BT, H))],
        out_shape=[jax.ShapeDtypeStruct((EH, 2 * H), _f32),
                   jax.ShapeDtypeStruct((EH, H), jnp.bfloat16)],
    )(ea_t, *args)


def _u_enc_body(u, W1, b1, W2, b2, g, b, Weu_e, Weu_h, b1e, Wxu_e, Wxu_h, b1x,
                o_encu, o_ue, o_ux):
    h = jnp.maximum(_dot(u[...], W1[...]) + b1[...], 0.0)
    eu = _ln(_dot(h, W2[...]) + b2[...], g[...], b[...])
    o_encu[...] = eu
    o_ue[...] = _dot(eu, Weu_e[...]) + _dot(eu, Weu_h[...]) + b1e[...]
    o_ux[...] = _dot(eu, Wxu_e[...]) + _dot(eu, Wxu_h[...]) + b1x[...]


def _u_enc_call(u, args):
    return pl.pallas_call(
        _u_enc_body,
        out_shape=[jax.ShapeDtypeStruct((1, H), _f32)] * 3,
    )(u, *args)


def _edge_core_body(S, C, P, ue, Weh, W2, b2, g, b, o_heP, o_esum):
    h1 = S[...] + C[...].astype(_f32) + P[...][:, H:] + ue[...]
    out = _ln(_dot(jnp.maximum(h1, 0.0), W2[...]) + b2[...], g[...], b[...])
    o_heP[...] = jnp.concatenate([out, _dot(out, Weh[...])], axis=-1)

    @pl.when(pl.program_id(0) == 0)
    def _():
        o_esum[...] = jnp.zeros_like(o_esum)

    o_esum[...] += jnp.sum(out, axis=0, keepdims=True)


def _edge_core_call(S, C, heP, ue, args):
    return pl.pallas_call(
        _edge_core_body,
        grid=(EH // BE,),
        in_specs=[_rows((BE, H)), _rows((BE, H))]
        + [_rows((BE, 2 * H)), _full((1, H))]
        + [_full(a.shape) for a in args],
        out_specs=[_rows((BE, 2 * H)), _full((1, H))],
        out_shape=[jax.ShapeDtypeStruct((EH, 2 * H), _f32),
                   jax.ShapeDtypeStruct((1, H), _f32)],
    )(S, C, heP, ue, *args)


def _node_core_body(ns, ns1, cnt, hx, Nx, Ax, Bx, ux, Wxh, Wxa, W2, b2, g, b,
                    Wsh, Wdh, o_hx, o_T, o_xsum):
    s = ns[0, :, :H] + ns[1, :, :H] + ns1[0, :, :H] + ns1[1, :, :H]
    c = cnt[0, :, 0:1] + cnt[1, :, 0:1]
    eagg = s / jnp.maximum(c, 1.0)
    h1 = Nx[...] + _dot(hx[...], Wxh[...]) + _dot(eagg, Wxa[...]) + ux[...]
    hxn = _ln(_dot(jnp.maximum(h1, 0.0), W2[...]) + b2[...], g[...], b[...])
    o_hx[...] = hxn
    o_T[...] = jnp.concatenate(
        [Ax[...] + _dot(hxn, Wsh[...]), Bx[...] + _dot(hxn, Wdh[...])],
        axis=-1)

    @pl.when(pl.program_id(0) == 0)
    def _():
        o_xsum[...] = jnp.zeros_like(o_xsum)

    o_xsum[...] += jnp.sum(hxn, axis=0, keepdims=True)


def _node_core_call(ns, ns1, cnt, hx, Nx, Ax, Bx, ux, args):
    return pl.pallas_call(
        _node_core_body,
        grid=(N // BN,),
        in_specs=[pl.BlockSpec((2, BN, 2 * H), lambda i: (0, i, 0)),
                  pl.BlockSpec((2, BN, 2 * H), lambda i: (0, i, 0)),
                  pl.BlockSpec((2, BN, 128), lambda i: (0, i, 0))]
        + [_rows((BN, H))] * 4 + [_full((1, H))]
        + [_full(a.shape) for a in args],
        out_specs=[_rows((BN, H)), _rows((BN, 2 * H)), _full((1, H))],
        out_shape=[jax.ShapeDtypeStruct((N, H), _f32),
                   jax.ShapeDtypeStruct((N, 2 * H), _f32),
                   jax.ShapeDtypeStruct((1, H), _f32)],
    )(ns, ns1, cnt, hx, Nx, Ax, Bx, ux, *args)


def _global_body(xsum, esum, esum1, encu, hu, Wu_x, Wu_e, Wu_ue, Wu_uh, b1u,
                 W2u, b2u, gu, bu, Weu_e, Weu_h, b1e, Wxu_e, Wxu_h, b1x,
                 Wd1, bd1, Wd2, bd2, gd, bd,
                 o_hu, o_ue, o_ux, o_uout):
    xa = xsum[...] * (1.0 / N)
    ea = (esum[...] + esum1[...]) * (1.0 / E)
    h1 = (_dot(xa, Wu_x[...]) + _dot(ea, Wu_e[...]) + _dot(encu[...], Wu_ue[...])
          + _dot(hu[...], Wu_uh[...]) + b1u[...])
    hun = _ln(_dot(jnp.maximum(h1, 0.0), W2u[...]) + b2u[...], gu[...], bu[...])
    o_hu[...] = hun
    o_ue[...] = _dot(encu[...], Weu_e[...]) + _dot(hun, Weu_h[...]) + b1e[...]
    o_ux[...] = _dot(encu[...], Wxu_e[...]) + _dot(hun, Wxu_h[...]) + b1x[...]
    hd = jnp.maximum(_dot(hun, Wd1[...]) + bd1[...], 0.0)
    o_uout[...] = _ln(_dot(hd, Wd2[...]) + bd2[...], gd[...], bd[...])


def _global_call(xsum, esum, esum1, encu, hu, args):
    return pl.pallas_call(
        _global_body,
        out_shape=[jax.ShapeDtypeStruct((1, H), _f32)] * 3
        + [jax.ShapeDtypeStruct((1, 16), _f32)],
    )(xsum, esum, esum1, encu, hu, *args)


def _dec_body(h, W1, b1, W2, b2, g, b, o):
    a = jnp.maximum(_dot(h[...][:, :H], W1[...]) + b1[...], 0.0)
    o[...] = _ln(_dot(a, W2[...]) + b2[...], g[...], b[...])


def _dec_e_body(h, W1, b1, W2, b2, g, b, o):
    a = jnp.maximum(_dot(h[...][:, :H], W1[...]) + b1[...], 0.0)
    ot = lax.dot_general(W2[...], a, (((0,), (1,)), ((), ())),
                         preferred_element_type=_f32) + b2[...]
    mu = jnp.mean(ot, axis=0, keepdims=True)
    d = ot - mu
    var = jnp.mean(d * d, axis=0, keepdims=True)
    o[...] = d / jnp.sqrt(var + 1e-5) * g[...] + b[...]


def _dec_e_call(h, args):
    return pl.pallas_call(
        _dec_e_body,
        grid=(EH // BT,),
        in_specs=[_rows((BT, 2 * H))] + [_full(a.shape) for a in args],
        out_specs=pl.BlockSpec((16, BT), lambda i: (0, i)),
        out_shape=jax.ShapeDtypeStruct((16, EH), _f32),
    )(h, *args)


def _dec_call(h, args, rows, brows, fout):
    return pl.pallas_call(
        _dec_body,
        grid=(rows // brows,),
        in_specs=[_rows((brows, h.shape[1]))] + [_full(a.shape) for a in args],
        out_specs=_rows((brows, fout)),
        out_shape=jax.ShapeDtypeStruct((rows, fout), _f32),
    )(h, *args)


def _mesh():
    return plsc.VectorSubcoreMesh(core_axis_name="c", subcore_axis_name="s",
                                  num_cores=NC, num_subcores=NS)


@functools.lru_cache(maxsize=None)
def _sc_gather_kernel(etot, ew, nch, ch):
    def body(t_h, row_h, col_h, s_h, idxa, idxb, ba0, bb0, ba1, bb1,
             sb0, sb1, ga0, gb0, ga1, gb1, w0, w1):
        c = lax.axis_index("c")
        s = lax.axis_index("s")
        w = c * NS + s
        pltpu.sync_copy(row_h.at[w], idxa)
        pltpu.sync_copy(col_h.at[w], idxb)

        def start(j, bufa, bufb, sa, sb):
            pltpu.async_copy(t_h.at[idxa.at[j]], bufa, sa)
            pltpu.async_copy(t_h.at[idxb.at[j]], bufb, sb)

        def finish(j, bufa, bufb, sa, sb, sbuf, ws):
            pltpu.make_async_copy(t_h.at[pl.ds(0, ch)], bufa, sa).wait()
            pltpu.make_async_copy(t_h.at[pl.ds(0, ch)], bufb, sb).wait()

            @pl.when(j >= 2)
            def _():
                pltpu.make_async_copy(sbuf, s_h.at[pl.ds(0, ch), :], ws).wait()

            def add(r, carry):
                for l in range(H // 16):
                    va = bufa[r, pl.ds(l * 16, 16)]
                    vb = bufb[r, pl.ds(H + l * 16, 16)]
                    sbuf[r, pl.ds(l * 16, 16)] = va + vb
                return carry

            lax.fori_loop(0, ch, add, 0)
            pltpu.async_copy(sbuf, s_h.at[pl.ds(w * ew + j * ch, ch), :], ws)

        start(0, ba0, bb0, ga0, gb0)

        def loop(k, carry):
            j0 = 2 * k
            start(j0 + 1, ba1, bb1, ga1, gb1)
            finish(j0, ba0, bb0, ga0, gb0, sb0, w0)

            @pl.when(j0 + 2 < nch)
            def _():
                start(j0 + 2, ba0, bb0, ga0, gb0)

            finish(j0 + 1, ba1, bb1, ga1, gb1, sb1, w1)
            return carry

        lax.fori_loop(0, nch // 2, loop, 0)
        finish(nch - 1, ba0, bb0, ga0, gb0, sb0, w0)
        pltpu.make_async_copy(sb0, s_h.at[pl.ds(0, ch), :], w0).wait()
        pltpu.make_async_copy(sb1, s_h.at[pl.ds(0, ch), :], w1).wait()

    return functools.partial(
        pl.kernel,
        out_type=jax.ShapeDtypeStruct((etot, H), _f32),
        mesh=_mesh(),
        scratch_types=[
            pltpu.VMEM((nch, ch), jnp.int32),
            pltpu.VMEM((nch, ch), jnp.int32),
            pltpu.VMEM((ch, 2 * H), _f32),
            pltpu.VMEM((ch, 2 * H), _f32),
            pltpu.VMEM((ch, 2 * H), _f32),
            pltpu.VMEM((ch, 2 * H), _f32),
            pltpu.VMEM((ch, H), _f32),
            pltpu.VMEM((ch, H), _f32),
        ] + [pltpu.SemaphoreType.DMA] * 6,
    )(body)


def _sc_gather(t, row2, col2):
    return _sc_gather_kernel(EH, EWH, NCH2, CH2)(t, row2, col2)


@functools.lru_cache(maxsize=None)
def _sc_scatter_kernel(ew, nch, ch):
    def body(he_h, col_h, zero_h, out_h, idx, buf0, buf1, acc, l0, l1, s0, s1):
        c = lax.axis_index("c")
        s = lax.axis_index("s")
        w = c * NS + s

        @pl.when(s == 0)
        def _():
            pltpu.sync_copy(zero_h, acc)

        pltpu.sync_copy(col_h.at[w], idx)

        def load(j, buf, ls, ss):
            @pl.when(j >= 2)
            def _():
                pltpu.make_async_copy(buf, acc.at[idx.at[0]], ss).wait()

            pltpu.async_copy(he_h.at[pl.ds(w * ew + j * ch, ch), :], buf, ls)

        def process(j, buf, ls, ss):
            pltpu.make_async_copy(he_h.at[pl.ds(0, ch), :], buf, ls).wait()
            pltpu.make_async_copy(buf, acc.at[idx.at[j]], ss).start(add=True)

        load(0, buf0, l0, s0)
        plsc.subcore_barrier()

        def loop(k, carry):
            j0 = 2 * k
            load(j0 + 1, buf1, l1, s1)
            process(j0, buf0, l0, s0)

            @pl.when(j0 + 2 < nch)
            def _():
                load(j0 + 2, buf0, l0, s0)

            process(j0 + 1, buf1, l1, s1)
            return carry

        lax.fori_loop(0, nch // 2, loop, 0)
        process(nch - 1, buf0, l0, s0)
        pltpu.make_async_copy(buf0, acc.at[idx.at[0]], s0).wait()
        pltpu.make_async_copy(buf1, acc.at[idx.at[0]], s1).wait()
        plsc.subcore_barrier()
        pltpu.sync_copy(acc.at[pl.ds(s * NPT, NPT), :], out_h.at[c, s])

    return functools.partial(
        pl.kernel,
        out_type=jax.ShapeDtypeStruct((NC, NS, NPT, 2 * H), _f32),
        mesh=_mesh(),
        scratch_types=[
            pltpu.VMEM((nch, ch), jnp.int32),
            pltpu.VMEM((ch, 2 * H), _f32),
            pltpu.VMEM((ch, 2 * H), _f32),
            pltpu.VMEM_SHARED((NPAD, 2 * H), _f32),
        ] + [pltpu.SemaphoreType.DMA] * 4,
    )(body)


def _sc_scatter(he, col2, zeros_nh):
    return _sc_scatter_kernel(EWH, NCH2, CH2)(he, col2, zeros_nh)


@functools.lru_cache(maxsize=None)
def _sc_count_kernel():
    return functools.partial(
        pl.kernel,
        out_type=jax.ShapeDtypeStruct((NC, NS, NPT, 128), _f32),
        mesh=_mesh(),
        scratch_types=[
            pltpu.VMEM((NCH, CH), jnp.int32),
            pltpu.VMEM((CH, 128), _f32),
            pltpu.VMEM_SHARED((NPAD, 128), _f32),
        ],
    )(_sc_count_body)


def _sc_count(col2, ones, zeros):
    return _sc_count_kernel()(col2, ones, zeros)


def _sc_count_body(col_h, ones_h, zero_h, out_h, idx, buf, acc):
    c = lax.axis_index("c")
    s = lax.axis_index("s")
    w = c * NS + s

    @pl.when(s == 0)
    def _():
        pltpu.sync_copy(zero_h, acc)

    pltpu.sync_copy(col_h.at[w], idx)
    pltpu.sync_copy(ones_h, buf)
    plsc.subcore_barrier()

    def chunk(j, carry):
        pltpu.sync_copy(buf, acc.at[idx.at[j]], add=True)
        return carry

    lax.fori_loop(0, NCH, chunk, 0)
    plsc.subcore_barrier()
    pltpu.sync_copy(acc.at[pl.ds(s * NPT, NPT), :], out_h.at[c, s])


def kernel(x, edge_index, edge_attr, u, batch, params):
    del batch
    row3 = edge_index[0].astype(jnp.int32).reshape(2, NW, NCH2, CH2)
    col3 = edge_index[1].astype(jnp.int32).reshape(2, NW, NCH2, CH2)
    col2 = edge_index[1].astype(jnp.int32).reshape(NW, NCH, CH)

    def vec(p, k):
        return p[k].reshape(1, -1)

    pe, px, pu = params["core_e"], params["core_x"], params["core_u"]
    We, Wx, Wu = pe["W1"], px["W1"], pu["W1"]
    Wes_e, Wes_h = We[0:64], We[64:128]
    Wed_e, Wed_h = We[128:192], We[192:256]
    Wee_e, Wee_h = We[256:320], We[320:384]
    Weu_e, Weu_h = We[384:448], We[448:512]
    Wxx_e, Wxx_h = Wx[0:64], Wx[64:128]
    Wxa = Wx[128:192]
    Wxu_e, Wxu_h = Wx[192:256], Wx[256:320]
    Wu_x, Wu_e = Wu[0:64], Wu[64:128]
    Wu_ue, Wu_uh = Wu[128:192], Wu[192:256]
    b1e = vec(pe, "b1")
    b1x = vec(px, "b1")

    p = params["enc_x"]
    enc_x, T, Ax, Bx, Nx = _enc_node_call(
        x, [p["W1"], vec(p, "b1"), p["W2"], vec(p, "b2"), vec(p, "ln_g"),
            vec(p, "ln_b"), Wes_e, Wes_h, Wed_e, Wed_h, Wxx_e])
    p = params["enc_e"]
    enc_e_args = [p["W1"], vec(p, "b1"), p["W2"], vec(p, "b2"),
                  vec(p, "ln_g"), vec(p, "ln_b"), Wee_e, Wee_h]
    heP0, C0 = _enc_edge_call(edge_attr.T, enc_e_args, 0)
    heP1, C1 = _enc_edge_call(edge_attr.T, enc_e_args, 1)
    p = params["enc_u"]
    enc_u, ue, ux = _u_enc_call(
        u, [p["W1"], vec(p, "b1"), p["W2"], vec(p, "b2"), vec(p, "ln_g"),
            vec(p, "ln_b"), Weu_e, Weu_h, b1e, Wxu_e, Wxu_h, b1x])

    cnt = _sc_count(col2, jnp.ones((CH, 128), _f32),
                    jnp.zeros((NPAD, 128), _f32)).reshape(NC, NPAD, 128)
    zeros_nh = jnp.zeros((NPAD, 2 * H), _f32)

    hx, hu = enc_x, enc_u
    pd = params["dec_u"]
    u_out = None
    ec_args = [Wee_h, pe["W2"], vec(pe, "b2"), vec(pe, "ln_g"),
               vec(pe, "ln_b")]
    for _ in range(NSTEPS):
        S0 = _sc_gather(T, row3[0], col3[0])
        heP0, esum0 = _edge_core_call(S0, C0, heP0, ue, ec_args)
        S1 = _sc_gather(T, row3[1], col3[1])
        heP1, esum1 = _edge_core_call(S1, C1, heP1, ue, ec_args)
        ns0 = _sc_scatter(heP0, col3[0], zeros_nh).reshape(NC, NPAD, 2 * H)
        ns1 = _sc_scatter(heP1, col3[1], zeros_nh).reshape(NC, NPAD, 2 * H)
        hx, T, xsum = _node_core_call(
            ns0, ns1, cnt, hx, Nx, Ax, Bx, ux,
            [Wxx_h, Wxa, px["W2"], vec(px, "b2"), vec(px, "ln_g"),
             vec(px, "ln_b"), Wes_h, Wed_h])
        hu, ue, ux, u_out = _global_call(
            xsum, esum0, esum1, enc_u, hu,
            [Wu_x, Wu_e, Wu_ue, Wu_uh, vec(pu, "b1"), pu["W2"], vec(pu, "b2"),
             vec(pu, "ln_g"), vec(pu, "ln_b"), Weu_e, Weu_h, b1e, Wxu_e,
             Wxu_h, b1x, pd["W1"], vec(pd, "b1"), pd["W2"], vec(pd, "b2"),
             vec(pd, "ln_g"), vec(pd, "ln_b")])

    p = params["dec_e"]
    dec_e_args = [p["W1"], vec(p, "b1"), p["W2"], p["b2"].reshape(-1, 1),
                  p["ln_g"].reshape(-1, 1), p["ln_b"].reshape(-1, 1)]
    e_out = jnp.concatenate(
        [_dec_e_call(heP0, dec_e_args), _dec_e_call(heP1, dec_e_args)],
        axis=1).T
    p = params["dec_x"]
    x_out = _dec_call(hx, [p["W1"], vec(p, "b1"), p["W2"], vec(p, "b2"),
                           vec(p, "ln_g"), vec(p, "ln_b")], N, BN, 128)
    return (e_out, x_out, u_out)
